# baseline (speedup 1.0000x reference)
import functools

import jax
import jax.numpy as jnp
from jax import lax
from jax.experimental import pallas as pl
from jax.experimental.pallas import tpu as pltpu
from jax.experimental.pallas import tpu_sc as plsc

N = 10000
E = 160000
D = 256
DH = 128
DQ = 64
NT = 64
NC = 2
NS = 16
BN = 2000
GRID = N // BN

EPW = E // NS
WIN = 128
NFULL = EPW // WIN
TAIL = EPW - NFULL * WIN
ZR = 400


@functools.cache
def _sc_mesh():
    return plsc.VectorSubcoreMesh(
        core_axis_name="c", subcore_axis_name="s", num_cores=NC, num_subcores=NS
    )


def _deg_body(ei_hbm, zd_hbm, ones_hbm, out_hbm, idx_v, tidx_v, ones_v, zst_v,
              deg_sh, semi, sems):
    c = lax.axis_index("c")
    s = lax.axis_index("s")

    pltpu.sync_copy(zd_hbm, zst_v)
    pltpu.sync_copy(ones_hbm, ones_v)

    @pl.when(s < 10)
    def _():
        pltpu.sync_copy(zst_v, deg_sh.at[pl.ds(s * 1000, 1000)])

    plsc.subcore_barrier()

    def load(k, b):
        base = pl.multiple_of(c * E + s * EPW + k * WIN, 8)
        pltpu.async_copy(ei_hbm.at[pl.ds(base, WIN)], idx_v.at[b],
                         semi.at[b])

    def load_wait(b):
        pltpu.make_async_copy(ei_hbm.at[pl.ds(0, WIN)], idx_v.at[b],
                              semi.at[b]).wait()

    def scat_issue(b):
        pltpu.async_copy(ones_v, deg_sh.at[idx_v.at[b]], sems.at[b],
                         add=True)

    def scat_wait(b):
        pltpu.make_async_copy(ones_v, deg_sh.at[idx_v.at[b]],
                              sems.at[b]).wait()

    load(0, 0)
    load(1, 1)

    def triple(kk, carry):
        for b in (0, 1, 2):
            k = 3 * kk + b
            sp = (k + 2) % 3
            load_wait(b)
            scat_issue(b)
            if b == 0:
                pl.when(kk > 0)(lambda: scat_wait(sp))
                load(k + 2, sp)
            else:
                pl.when(kk < NFULL // 3 - 1)(lambda: scat_wait(sp))
                pl.when(kk < NFULL // 3 - 1)(lambda: load(k + 2, sp))
        return carry

    lax.fori_loop(0, NFULL // 3, triple, 0)
    scat_wait(0)
    scat_wait(1)
    scat_wait(2)
    tbase = pl.multiple_of(c * E + s * EPW + NFULL * WIN, 8)
    pltpu.sync_copy(ei_hbm.at[pl.ds(tbase, TAIL)], tidx_v.at[0])
    pltpu.sync_copy(ones_v.at[pl.ds(0, TAIL)],
                    deg_sh.at[tidx_v.at[0]], add=True)
    plsc.subcore_barrier()

    @pl.when(s < 10)
    def _():
        pltpu.sync_copy(deg_sh.at[pl.ds(s * 1000, 1000)], zst_v)
        ob = pl.multiple_of(c * N + s * 1000, 8)
        pltpu.sync_copy(zst_v, out_hbm.at[pl.ds(ob, 1000)])


@jax.jit
def _sc_degrees(edge_index):
    zd = jnp.zeros((1000,), jnp.float32)
    ones = jnp.ones((WIN,), jnp.float32)
    f = pl.kernel(
        _deg_body,
        out_type=jax.ShapeDtypeStruct((NC * N,), jnp.float32),
        mesh=_sc_mesh(),
        scratch_types=[
            pltpu.VMEM((3, WIN), jnp.int32),
            pltpu.VMEM((1, TAIL), jnp.int32),
            pltpu.VMEM((WIN,), jnp.float32),
            pltpu.VMEM((1000,), jnp.float32),
            pltpu.VMEM_SHARED((N,), jnp.float32),
            pltpu.SemaphoreType.DMA((3,)),
            pltpu.SemaphoreType.DMA((3,)),
        ],
        name="gcn_degrees",
    )
    return f(edge_index.reshape(2 * E), zd, ones)


def _trash_vec(i):
    return N + ((lax.iota(jnp.int32, 16) + i * 16) & (NT - 1))


def _scat_body(p_hbm, ei_hbm, z_hbm, out_hbm, sbuf, dbuf, rows_v, zst_v,
               m_sh, semg, semi, semd, sems):
    c = lax.axis_index("c")
    s = lax.axis_index("s")
    ebase = s * EPW

    def src_load(k, b):
        pltpu.async_copy(ei_hbm.at[pl.ds(pl.multiple_of(ebase + k * WIN, 8),
                                         WIN)],
                         sbuf.at[b], semi.at[b])

    def src_wait(b):
        pltpu.make_async_copy(ei_hbm.at[pl.ds(0, WIN)], sbuf.at[b],
                              semi.at[b]).wait()

    def add_off(b, off):
        for i in range(WIN // 16):
            sl = pl.ds(i * 16, 16)
            sbuf[b, sl] = sbuf[b, sl] + off

    def gather_issue(b):
        pltpu.async_copy(p_hbm.at[sbuf.at[b]], rows_v.at[b], semg.at[b])

    def gather_wait(b):
        pltpu.make_async_copy(p_hbm.at[sbuf.at[b]], rows_v.at[b],
                              semg.at[b]).wait()

    def dst_load(k, b):
        pltpu.async_copy(ei_hbm.at[pl.ds(pl.multiple_of(E + ebase + k * WIN,
                                                        8), WIN)],
                         dbuf.at[b], semd.at[b])

    def dst_wait(b):
        pltpu.make_async_copy(ei_hbm.at[pl.ds(0, WIN)], dbuf.at[b],
                              semd.at[b]).wait()

    def scat_issue(b):
        pltpu.async_copy(rows_v.at[b], m_sh.at[dbuf.at[b]], sems.at[b],
                         add=True)

    def scat_wait(b):
        pltpu.make_async_copy(rows_v.at[b], m_sh.at[dbuf.at[b]],
                              sems.at[b]).wait()

    for qq in range(2):
        off = (c * 2 + qq) * N

        pltpu.sync_copy(z_hbm, zst_v)

        @pl.when(s < 10)
        def _():
            for j in (0, ZR, 2 * ZR):
                n = min(ZR, 1000 - j)
                pltpu.sync_copy(zst_v.at[pl.ds(0, n)],
                                m_sh.at[pl.ds(s * 1000 + j, n)])

        @pl.when(s == 10)
        def _():
            pltpu.sync_copy(zst_v.at[pl.ds(0, NT)], m_sh.at[pl.ds(N, NT)])

        plsc.subcore_barrier()

        src_load(0, 0)
        src_wait(0)
        add_off(0, off)
        gather_issue(0)
        dst_load(0, 0)
        src_load(1, 1)

        def pair(kk, carry):
            for b in (0, 1):
                k = 2 * kk + b
                later = kk < NFULL // 2 - 1

                if b == 0:
                    pl.when(kk > 0)(lambda: scat_wait(1))
                else:
                    scat_wait(0)

                def issue_next():
                    src_wait(1 - b)
                    add_off(1 - b, off)
                    gather_issue(1 - b)

                if b == 0:
                    issue_next()
                else:
                    pl.when(later)(issue_next)
                gather_wait(b)
                pl.when(later)(lambda: src_load(k + 2, b))
                dst_wait(b)
                if b == 0:
                    dst_load(k + 1, 1)
                else:
                    pl.when(later)(lambda: dst_load(k + 1, 0))
                scat_issue(b)
            return carry

        lax.fori_loop(0, NFULL // 2, pair, 0)

        tb = pl.multiple_of(ebase + NFULL * WIN, 8)
        pltpu.sync_copy(ei_hbm.at[pl.ds(tb, TAIL)],
                        sbuf.at[0, pl.ds(0, TAIL)])
        for i in range(TAIL // 16, WIN // 16):
            sbuf[0, pl.ds(i * 16, 16)] = lax.iota(jnp.int32, 16) + i * 16
        add_off(0, off)
        pltpu.async_copy(p_hbm.at[sbuf.at[0]], rows_v.at[0], semg.at[0])
        gather_wait(0)
        tdb = pl.multiple_of(E + tb, 8)
        pltpu.sync_copy(ei_hbm.at[pl.ds(tdb, TAIL)],
                        dbuf.at[0, pl.ds(0, TAIL)])
        for i in range(TAIL // 16, WIN // 16):
            dbuf[0, pl.ds(i * 16, 16)] = _trash_vec(i)
        scat_wait(1)
        scat_issue(0)
        scat_wait(0)
        plsc.subcore_barrier()

        @pl.when(s < 10)
        def _():
            for j in (0, ZR, 2 * ZR):
                n = min(ZR, 1000 - j)
                pltpu.sync_copy(m_sh.at[pl.ds(s * 1000 + j, n)],
                                zst_v.at[pl.ds(0, n)])
                pltpu.sync_copy(zst_v.at[pl.ds(0, n)],
                                out_hbm.at[pl.ds(off + s * 1000 + j, n)])

        plsc.subcore_barrier()


@jax.jit
def _sc_scatter(p_flat, edge_index):
    z = jnp.zeros((ZR, DQ), jnp.float32)
    f = pl.kernel(
        _scat_body,
        out_type=jax.ShapeDtypeStruct((4 * N, DQ), jnp.float32),
        mesh=_sc_mesh(),
        scratch_types=[
            pltpu.VMEM((2, WIN), jnp.int32),
            pltpu.VMEM((2, WIN), jnp.int32),
            pltpu.VMEM((2, WIN, DQ), jnp.float32),
            pltpu.VMEM((ZR, DQ), jnp.float32),
            pltpu.VMEM_SHARED((N + NT, DQ), jnp.float32),
            pltpu.SemaphoreType.DMA((2,)),
            pltpu.SemaphoreType.DMA((2,)),
            pltpu.SemaphoreType.DMA((2,)),
            pltpu.SemaphoreType.DMA((2,)),
        ],
        compiler_params=pltpu.CompilerParams(use_tc_tiling_on_sc=False),
        name="gcn_edge_scatter",
    )
    return f(p_flat, edge_index.reshape(2 * E), z)


BH = BN // 2


def _csc(deg_ref):
    return lax.rsqrt(jnp.maximum(deg_ref[...], 1.0))


def _split_store(p_ref, ql, qr):
    for i in range(4):
        sl = slice(i * DQ, (i + 1) * DQ)
        p_ref[i] = jnp.concatenate([ql[:, sl], qr[:, sl]], axis=-1)


def _munpack(m_ref):
    ml = jnp.concatenate([m_ref[i][:, :DQ] for i in range(4)], axis=-1)
    mr = jnp.concatenate([m_ref[i][:, DQ:] for i in range(4)], axis=-1)
    return ml, mr


def _pre_body(xl_ref, xr_ref, w_ref, rw_ref, rb_ref, dsl_ref, dsr_ref,
              p_ref, rl_ref, rr_ref):
    ql = jnp.dot(xl_ref[...], w_ref[...], preferred_element_type=jnp.float32)
    qr = jnp.dot(xr_ref[...], w_ref[...], preferred_element_type=jnp.float32)
    _split_store(p_ref, ql * _csc(dsl_ref), qr * _csc(dsr_ref))
    rl_ref[...] = (
        jnp.dot(xl_ref[...], rw_ref[...], preferred_element_type=jnp.float32)
        + rb_ref[...]
    )
    rr_ref[...] = (
        jnp.dot(xr_ref[...], rw_ref[...], preferred_element_type=jnp.float32)
        + rb_ref[...]
    )


def _mid_body(m_ref, ddl_ref, ddr_ref, b_ref, rl_ref, rr_ref, w_ref, rw_ref,
              rb_ref, dsl_ref, dsr_ref, p_ref, r2l_ref, r2r_ref):
    ml, mr = _munpack(m_ref)
    hl = jax.nn.relu(_csc(ddl_ref) * ml + b_ref[...]) + rl_ref[...]
    hr = jax.nn.relu(_csc(ddr_ref) * mr + b_ref[...]) + rr_ref[...]
    ql = jnp.dot(hl, w_ref[...], preferred_element_type=jnp.float32)
    qr = jnp.dot(hr, w_ref[...], preferred_element_type=jnp.float32)
    _split_store(p_ref, ql * _csc(dsl_ref), qr * _csc(dsr_ref))
    r2l_ref[...] = (
        jnp.dot(hl, rw_ref[...], preferred_element_type=jnp.float32)
        + rb_ref[...]
    )
    r2r_ref[...] = (
        jnp.dot(hr, rw_ref[...], preferred_element_type=jnp.float32)
        + rb_ref[...]
    )


def _mid2_body(m_ref, ddl_ref, ddr_ref, b_ref, rl_ref, rr_ref, w_ref,
               dsl_ref, dsr_ref, p_ref, hl_ref, hr_ref):
    ml, mr = _munpack(m_ref)
    hl = jax.nn.relu(_csc(ddl_ref) * ml + b_ref[...]) + rl_ref[...]
    hr = jax.nn.relu(_csc(ddr_ref) * mr + b_ref[...]) + rr_ref[...]
    ql = jnp.dot(hl, w_ref[...], preferred_element_type=jnp.float32)
    qr = jnp.dot(hr, w_ref[...], preferred_element_type=jnp.float32)
    _split_store(p_ref, ql * _csc(dsl_ref), qr * _csc(dsr_ref))
    hl_ref[...] = hl
    hr_ref[...] = hr


def _post_body(m_ref, ddl_ref, ddr_ref, b_ref, hl_ref, hr_ref,
               outl_ref, outr_ref):
    ml, mr = _munpack(m_ref)
    outl_ref[...] = _csc(ddl_ref) * ml + b_ref[...] + hl_ref[...]
    outr_ref[...] = _csc(ddr_ref) * mr + b_ref[...] + hr_ref[...]


_ROW = pl.BlockSpec((BH, D), lambda i: (i, 0))
_COL = pl.BlockSpec((BH, 1), lambda i: (i, 0))
_WMAT = pl.BlockSpec((D, D), lambda i: (0, 0))
_BVEC = pl.BlockSpec((1, D), lambda i: (0, 0))
_MIN = pl.BlockSpec((4, BH, 2 * DQ), lambda i: (0, i, 0))

_P_OUT = jax.ShapeDtypeStruct((4, N // 2, 2 * DQ), jnp.float32)
_R_OUT = jax.ShapeDtypeStruct((N // 2, D), jnp.float32)


@jax.jit
def _tc_pre(xl, xr, w, rw, rb, dsl, dsr):
    return pl.pallas_call(
        _pre_body,
        grid=(GRID,),
        in_specs=[_ROW, _ROW, _WMAT, _WMAT, _BVEC, _COL, _COL],
        out_specs=[_MIN, _ROW, _ROW],
        out_shape=[_P_OUT, _R_OUT, _R_OUT],
    )(xl, xr, w, rw, rb, dsl, dsr)


@jax.jit
def _tc_mid(m, ddl, ddr, b, rl, rr, w, rw, rb, dsl, dsr):
    return pl.pallas_call(
        _mid_body,
        grid=(GRID,),
        in_specs=[_MIN, _COL, _COL, _BVEC, _ROW, _ROW, _WMAT, _WMAT, _BVEC,
                  _COL, _COL],
        out_specs=[_MIN, _ROW, _ROW],
        out_shape=[_P_OUT, _R_OUT, _R_OUT],
    )(m, ddl, ddr, b, rl, rr, w, rw, rb, dsl, dsr)


@jax.jit
def _tc_mid2(m, ddl, ddr, b, rl, rr, w, dsl, dsr):
    return pl.pallas_call(
        _mid2_body,
        grid=(GRID,),
        in_specs=[_MIN, _COL, _COL, _BVEC, _ROW, _ROW, _WMAT, _COL, _COL],
        out_specs=[_MIN, _ROW, _ROW],
        out_shape=[_P_OUT, _R_OUT, _R_OUT],
    )(m, ddl, ddr, b, rl, rr, w, dsl, dsr)


@jax.jit
def _tc_post(m, ddl, ddr, b, hl, hr):
    return pl.pallas_call(
        _post_body,
        grid=(GRID,),
        in_specs=[_MIN, _COL, _COL, _BVEC, _ROW, _ROW],
        out_specs=[_ROW, _ROW],
        out_shape=[_R_OUT, _R_OUT],
    )(m, ddl, ddr, b, hl, hr)


def kernel(x, edge_index, W1, b1, W2, b2, W3, b3, rW1, rb1, rW2, rb2):
    deg = _sc_degrees(edge_index)
    dsl = deg[:N:2].reshape(N // 2, 1)
    dsr = deg[1:N:2].reshape(N // 2, 1)
    ddl = deg[N::2].reshape(N // 2, 1)
    ddr = deg[N + 1::2].reshape(N // 2, 1)
    xl = x[0::2]
    xr = x[1::2]
    b1r = b1.reshape(1, D)
    b2r = b2.reshape(1, D)
    b3r = b3.reshape(1, D)
    rb1r = rb1.reshape(1, D)
    rb2r = rb2.reshape(1, D)

    def scat(p):
        m = _sc_scatter(p.reshape(4 * N, DQ), edge_index)
        return m.reshape(4, N // 2, 2 * DQ)

    p0, r0l, r0r = _tc_pre(xl, xr, W1, rW1, rb1r, dsl, dsr)
    m0 = scat(p0)
    p1, r1l, r1r = _tc_mid(m0, ddl, ddr, b1r, r0l, r0r, W2, rW2, rb2r,
                           dsl, dsr)
    m1 = scat(p1)
    p2, h2l, h2r = _tc_mid2(m1, ddl, ddr, b2r, r1l, r1r, W3, dsl, dsr)
    m2 = scat(p2)
    outl, outr = _tc_post(m2, ddl, ddr, b3r, h2l, h2r)
    return jnp.stack([outl, outr], axis=1).reshape(N, D)


# speedup vs baseline: 1.0541x; 1.0541x over previous
import functools

import jax
import jax.numpy as jnp
from jax import lax
from jax.experimental import pallas as pl
from jax.experimental.pallas import tpu as pltpu
from jax.experimental.pallas import tpu_sc as plsc

N = 10000
E = 160000
D = 256
DH = 128
DQ = 64
NT = 64
NC = 2
NS = 16
BN = 2000
GRID = N // BN

EPW = E // NS
WIN = 128
NFULL = EPW // WIN
TAIL = EPW - NFULL * WIN
ZR = 400


@functools.cache
def _sc_mesh():
    return plsc.VectorSubcoreMesh(
        core_axis_name="c", subcore_axis_name="s", num_cores=NC, num_subcores=NS
    )


def _deg_body(ei_hbm, zd_hbm, ones_hbm, out_hbm, idx_v, tidx_v, ones_v, zst_v,
              deg_sh, semi, sems):
    c = lax.axis_index("c")
    s = lax.axis_index("s")

    pltpu.sync_copy(zd_hbm, zst_v)
    pltpu.sync_copy(ones_hbm, ones_v)

    @pl.when(s < 10)
    def _():
        pltpu.sync_copy(zst_v, deg_sh.at[pl.ds(s * 1000, 1000)])

    plsc.subcore_barrier()

    def load(k, b):
        base = pl.multiple_of(c * E + s * EPW + k * WIN, 8)
        pltpu.async_copy(ei_hbm.at[pl.ds(base, WIN)], idx_v.at[b],
                         semi.at[b])

    def load_wait(b):
        pltpu.make_async_copy(ei_hbm.at[pl.ds(0, WIN)], idx_v.at[b],
                              semi.at[b]).wait()

    def scat_issue(b):
        pltpu.async_copy(ones_v, deg_sh.at[idx_v.at[b]], sems.at[b],
                         add=True)

    def scat_wait(b):
        pltpu.make_async_copy(ones_v, deg_sh.at[idx_v.at[b]],
                              sems.at[b]).wait()

    load(0, 0)
    load(1, 1)

    def triple(kk, carry):
        for b in (0, 1, 2):
            k = 3 * kk + b
            sp = (k + 2) % 3
            load_wait(b)
            scat_issue(b)
            if b == 0:
                pl.when(kk > 0)(lambda: scat_wait(sp))
                load(k + 2, sp)
            else:
                pl.when(kk < NFULL // 3 - 1)(lambda: scat_wait(sp))
                pl.when(kk < NFULL // 3 - 1)(lambda: load(k + 2, sp))
        return carry

    lax.fori_loop(0, NFULL // 3, triple, 0)
    scat_wait(0)
    scat_wait(1)
    scat_wait(2)
    tbase = pl.multiple_of(c * E + s * EPW + NFULL * WIN, 8)
    pltpu.sync_copy(ei_hbm.at[pl.ds(tbase, TAIL)], tidx_v.at[0])
    pltpu.sync_copy(ones_v.at[pl.ds(0, TAIL)],
                    deg_sh.at[tidx_v.at[0]], add=True)
    plsc.subcore_barrier()

    @pl.when(s < 10)
    def _():
        pltpu.sync_copy(deg_sh.at[pl.ds(s * 1000, 1000)], zst_v)
        ob = pl.multiple_of(c * N + s * 1000, 8)
        pltpu.sync_copy(zst_v, out_hbm.at[pl.ds(ob, 1000)])


@jax.jit
def _sc_degrees(edge_index):
    zd = jnp.zeros((1000,), jnp.float32)
    ones = jnp.ones((WIN,), jnp.float32)
    f = pl.kernel(
        _deg_body,
        out_type=jax.ShapeDtypeStruct((NC * N,), jnp.float32),
        mesh=_sc_mesh(),
        scratch_types=[
            pltpu.VMEM((3, WIN), jnp.int32),
            pltpu.VMEM((1, TAIL), jnp.int32),
            pltpu.VMEM((WIN,), jnp.float32),
            pltpu.VMEM((1000,), jnp.float32),
            pltpu.VMEM_SHARED((N,), jnp.float32),
            pltpu.SemaphoreType.DMA((3,)),
            pltpu.SemaphoreType.DMA((3,)),
        ],
        name="gcn_degrees",
    )
    return f(edge_index.reshape(2 * E), zd, ones)


def _trash_vec(i):
    return N + ((lax.iota(jnp.int32, 16) + i * 16) & (NT - 1))


def _scat_body(p_hbm, ei_hbm, z_hbm, out_hbm, sbuf, dbuf, rows_v, zst_v,
               m_sh, semg, semi, semd, sems):
    c = lax.axis_index("c")
    s = lax.axis_index("s")
    ebase = s * EPW

    def src_load(k, b):
        pltpu.async_copy(ei_hbm.at[pl.ds(pl.multiple_of(ebase + k * WIN, 8),
                                         WIN)],
                         sbuf.at[b], semi.at[b])

    def src_wait(b):
        pltpu.make_async_copy(ei_hbm.at[pl.ds(0, WIN)], sbuf.at[b],
                              semi.at[b]).wait()

    def add_off(b, off):
        for i in range(WIN // 16):
            sl = pl.ds(i * 16, 16)
            sbuf[b, sl] = sbuf[b, sl] + off

    def gather_issue(b):
        pltpu.async_copy(p_hbm.at[sbuf.at[b]], rows_v.at[b], semg.at[b])

    def gather_wait(b):
        pltpu.make_async_copy(p_hbm.at[sbuf.at[b]], rows_v.at[b],
                              semg.at[b]).wait()

    def dst_load(k, b):
        pltpu.async_copy(ei_hbm.at[pl.ds(pl.multiple_of(E + ebase + k * WIN,
                                                        8), WIN)],
                         dbuf.at[b], semd.at[b])

    def dst_wait(b):
        pltpu.make_async_copy(ei_hbm.at[pl.ds(0, WIN)], dbuf.at[b],
                              semd.at[b]).wait()

    def scat_issue(b):
        pltpu.async_copy(rows_v.at[b], m_sh.at[dbuf.at[b]], sems.at[b],
                         add=True)

    def scat_wait(b):
        pltpu.make_async_copy(rows_v.at[b], m_sh.at[dbuf.at[b]],
                              sems.at[b]).wait()

    for qq in range(2):
        off = (c * 2 + qq) * N

        pltpu.sync_copy(z_hbm, zst_v)

        @pl.when(s < 10)
        def _():
            for j in (0, ZR, 2 * ZR):
                n = min(ZR, 1000 - j)
                pltpu.sync_copy(zst_v.at[pl.ds(0, n)],
                                m_sh.at[pl.ds(s * 1000 + j, n)])

        @pl.when(s == 10)
        def _():
            pltpu.sync_copy(zst_v.at[pl.ds(0, NT)], m_sh.at[pl.ds(N, NT)])

        plsc.subcore_barrier()

        src_load(0, 0)
        src_wait(0)
        add_off(0, off)
        gather_issue(0)
        dst_load(0, 0)
        src_load(1, 1)

        def pair(kk, carry):
            for b in (0, 1):
                k = 2 * kk + b
                later = kk < NFULL // 2 - 1

                if b == 0:
                    pl.when(kk > 0)(lambda: scat_wait(1))
                else:
                    scat_wait(0)

                def issue_next():
                    src_wait(1 - b)
                    add_off(1 - b, off)
                    gather_issue(1 - b)

                if b == 0:
                    issue_next()
                else:
                    pl.when(later)(issue_next)
                gather_wait(b)
                pl.when(later)(lambda: src_load(k + 2, b))
                dst_wait(b)
                if b == 0:
                    dst_load(k + 1, 1)
                else:
                    pl.when(later)(lambda: dst_load(k + 1, 0))
                scat_issue(b)
            return carry

        lax.fori_loop(0, NFULL // 2, pair, 0)

        tb = pl.multiple_of(ebase + NFULL * WIN, 8)
        pltpu.sync_copy(ei_hbm.at[pl.ds(tb, TAIL)],
                        sbuf.at[0, pl.ds(0, TAIL)])
        for i in range(TAIL // 16, WIN // 16):
            sbuf[0, pl.ds(i * 16, 16)] = lax.iota(jnp.int32, 16) + i * 16
        add_off(0, off)
        pltpu.async_copy(p_hbm.at[sbuf.at[0]], rows_v.at[0], semg.at[0])
        gather_wait(0)
        tdb = pl.multiple_of(E + tb, 8)
        pltpu.sync_copy(ei_hbm.at[pl.ds(tdb, TAIL)],
                        dbuf.at[0, pl.ds(0, TAIL)])
        for i in range(TAIL // 16, WIN // 16):
            dbuf[0, pl.ds(i * 16, 16)] = _trash_vec(i)
        scat_wait(1)
        scat_issue(0)
        scat_wait(0)
        plsc.subcore_barrier()

        @pl.when(s < 10)
        def _():
            for j in (0, ZR, 2 * ZR):
                n = min(ZR, 1000 - j)
                pltpu.sync_copy(m_sh.at[pl.ds(s * 1000 + j, n)],
                                zst_v.at[pl.ds(0, n)])
                pltpu.sync_copy(zst_v.at[pl.ds(0, n)],
                                out_hbm.at[pl.ds(off + s * 1000 + j, n)])

        plsc.subcore_barrier()


@jax.jit
def _sc_scatter(p_flat, edge_index):
    z = jnp.zeros((ZR, DQ), jnp.float32)
    f = pl.kernel(
        _scat_body,
        out_type=jax.ShapeDtypeStruct((4 * N, DQ), jnp.float32),
        mesh=_sc_mesh(),
        scratch_types=[
            pltpu.VMEM((2, WIN), jnp.int32),
            pltpu.VMEM((2, WIN), jnp.int32),
            pltpu.VMEM((2, WIN, DQ), jnp.float32),
            pltpu.VMEM((ZR, DQ), jnp.float32),
            pltpu.VMEM_SHARED((N + NT, DQ), jnp.float32),
            pltpu.SemaphoreType.DMA((2,)),
            pltpu.SemaphoreType.DMA((2,)),
            pltpu.SemaphoreType.DMA((2,)),
            pltpu.SemaphoreType.DMA((2,)),
        ],
        compiler_params=pltpu.CompilerParams(use_tc_tiling_on_sc=False),
        name="gcn_edge_scatter",
    )
    return f(p_flat, edge_index.reshape(2 * E), z)


BH = BN // 2


def _csc(deg_ref):
    return lax.rsqrt(jnp.maximum(deg_ref[...], 1.0))


def _split_store(p_ref, ql, qr):
    for i in range(4):
        sl = slice(i * DQ, (i + 1) * DQ)
        p_ref[i] = jnp.concatenate([ql[:, sl], qr[:, sl]], axis=-1)


def _munpack(m_ref):
    ml = jnp.concatenate([m_ref[i][:, :DQ] for i in range(4)], axis=-1)
    mr = jnp.concatenate([m_ref[i][:, DQ:] for i in range(4)], axis=-1)
    return ml, mr


def _pre_body(xl_ref, xr_ref, w_ref, rw_ref, rb_ref, dsl_ref, dsr_ref,
              p_ref, rl_ref, rr_ref):
    ql = jnp.dot(xl_ref[...], w_ref[...], preferred_element_type=jnp.float32)
    qr = jnp.dot(xr_ref[...], w_ref[...], preferred_element_type=jnp.float32)
    _split_store(p_ref, ql * _csc(dsl_ref), qr * _csc(dsr_ref))
    rl_ref[...] = (
        jnp.dot(xl_ref[...], rw_ref[...], preferred_element_type=jnp.float32)
        + rb_ref[...]
    )
    rr_ref[...] = (
        jnp.dot(xr_ref[...], rw_ref[...], preferred_element_type=jnp.float32)
        + rb_ref[...]
    )


def _mid_body(m_ref, ddl_ref, ddr_ref, b_ref, rl_ref, rr_ref, w_ref, rw_ref,
              rb_ref, dsl_ref, dsr_ref, p_ref, r2l_ref, r2r_ref):
    ml, mr = _munpack(m_ref)
    hl = jax.nn.relu(_csc(ddl_ref) * ml + b_ref[...]) + rl_ref[...]
    hr = jax.nn.relu(_csc(ddr_ref) * mr + b_ref[...]) + rr_ref[...]
    ql = jnp.dot(hl, w_ref[...], preferred_element_type=jnp.float32)
    qr = jnp.dot(hr, w_ref[...], preferred_element_type=jnp.float32)
    _split_store(p_ref, ql * _csc(dsl_ref), qr * _csc(dsr_ref))
    r2l_ref[...] = (
        jnp.dot(hl, rw_ref[...], preferred_element_type=jnp.float32)
        + rb_ref[...]
    )
    r2r_ref[...] = (
        jnp.dot(hr, rw_ref[...], preferred_element_type=jnp.float32)
        + rb_ref[...]
    )


def _mid2_body(m_ref, ddl_ref, ddr_ref, b_ref, rl_ref, rr_ref, w_ref,
               dsl_ref, dsr_ref, p_ref, hl_ref, hr_ref):
    ml, mr = _munpack(m_ref)
    hl = jax.nn.relu(_csc(ddl_ref) * ml + b_ref[...]) + rl_ref[...]
    hr = jax.nn.relu(_csc(ddr_ref) * mr + b_ref[...]) + rr_ref[...]
    ql = jnp.dot(hl, w_ref[...], preferred_element_type=jnp.float32)
    qr = jnp.dot(hr, w_ref[...], preferred_element_type=jnp.float32)
    _split_store(p_ref, ql * _csc(dsl_ref), qr * _csc(dsr_ref))
    hl_ref[...] = hl
    hr_ref[...] = hr


def _post_body(m_ref, ddl_ref, ddr_ref, b_ref, hl_ref, hr_ref, out_ref):
    ml, mr = _munpack(m_ref)
    out_ref[:, 0, :] = _csc(ddl_ref) * ml + b_ref[...] + hl_ref[...]
    out_ref[:, 1, :] = _csc(ddr_ref) * mr + b_ref[...] + hr_ref[...]


_ROW = pl.BlockSpec((BH, D), lambda i: (i, 0))
_COL = pl.BlockSpec((BH, 1), lambda i: (i, 0))
_WMAT = pl.BlockSpec((D, D), lambda i: (0, 0))
_BVEC = pl.BlockSpec((1, D), lambda i: (0, 0))
_MIN = pl.BlockSpec((4, BH, 2 * DQ), lambda i: (0, i, 0))

_P_OUT = jax.ShapeDtypeStruct((4, N // 2, 2 * DQ), jnp.float32)
_R_OUT = jax.ShapeDtypeStruct((N // 2, D), jnp.float32)


@jax.jit
def _tc_pre(xl, xr, w, rw, rb, dsl, dsr):
    return pl.pallas_call(
        _pre_body,
        grid=(GRID,),
        in_specs=[_ROW, _ROW, _WMAT, _WMAT, _BVEC, _COL, _COL],
        out_specs=[_MIN, _ROW, _ROW],
        out_shape=[_P_OUT, _R_OUT, _R_OUT],
    )(xl, xr, w, rw, rb, dsl, dsr)


@jax.jit
def _tc_mid(m, ddl, ddr, b, rl, rr, w, rw, rb, dsl, dsr):
    return pl.pallas_call(
        _mid_body,
        grid=(GRID,),
        in_specs=[_MIN, _COL, _COL, _BVEC, _ROW, _ROW, _WMAT, _WMAT, _BVEC,
                  _COL, _COL],
        out_specs=[_MIN, _ROW, _ROW],
        out_shape=[_P_OUT, _R_OUT, _R_OUT],
    )(m, ddl, ddr, b, rl, rr, w, rw, rb, dsl, dsr)


@jax.jit
def _tc_mid2(m, ddl, ddr, b, rl, rr, w, dsl, dsr):
    return pl.pallas_call(
        _mid2_body,
        grid=(GRID,),
        in_specs=[_MIN, _COL, _COL, _BVEC, _ROW, _ROW, _WMAT, _COL, _COL],
        out_specs=[_MIN, _ROW, _ROW],
        out_shape=[_P_OUT, _R_OUT, _R_OUT],
    )(m, ddl, ddr, b, rl, rr, w, dsl, dsr)


@jax.jit
def _tc_post(m, ddl, ddr, b, hl, hr):
    return pl.pallas_call(
        _post_body,
        grid=(GRID,),
        in_specs=[_MIN, _COL, _COL, _BVEC, _ROW, _ROW],
        out_specs=pl.BlockSpec((BH, 2, D), lambda i: (i, 0, 0)),
        out_shape=jax.ShapeDtypeStruct((N // 2, 2, D), jnp.float32),
    )(m, ddl, ddr, b, hl, hr)


def kernel(x, edge_index, W1, b1, W2, b2, W3, b3, rW1, rb1, rW2, rb2):
    deg = _sc_degrees(edge_index)
    dsl = deg[:N:2].reshape(N // 2, 1)
    dsr = deg[1:N:2].reshape(N // 2, 1)
    ddl = deg[N::2].reshape(N // 2, 1)
    ddr = deg[N + 1::2].reshape(N // 2, 1)
    xl = x[0::2]
    xr = x[1::2]
    b1r = b1.reshape(1, D)
    b2r = b2.reshape(1, D)
    b3r = b3.reshape(1, D)
    rb1r = rb1.reshape(1, D)
    rb2r = rb2.reshape(1, D)

    def scat(p):
        m = _sc_scatter(p.reshape(4 * N, DQ), edge_index)
        return m.reshape(4, N // 2, 2 * DQ)

    p0, r0l, r0r = _tc_pre(xl, xr, W1, rW1, rb1r, dsl, dsr)
    m0 = scat(p0)
    p1, r1l, r1r = _tc_mid(m0, ddl, ddr, b1r, r0l, r0r, W2, rW2, rb2r,
                           dsl, dsr)
    m1 = scat(p1)
    p2, h2l, h2r = _tc_mid2(m1, ddl, ddr, b2r, r1l, r1r, W3, dsl, dsr)
    m2 = scat(p2)
    out = _tc_post(m2, ddl, ddr, b3r, h2l, h2r)
    return out.reshape(N, D)


# speedup vs baseline: 1.0923x; 1.0362x over previous
import functools

import jax
import jax.numpy as jnp
from jax import lax
from jax.experimental import pallas as pl
from jax.experimental.pallas import tpu as pltpu
from jax.experimental.pallas import tpu_sc as plsc

N = 10000
E = 160000
D = 256
DH = 128
DQ = 64
NT = 64
NC = 2
NS = 16
BN = 2000
GRID = N // BN

EPW = E // NS
WIN = 128
NFULL = EPW // WIN
TAIL = EPW - NFULL * WIN
ZR = 400


@functools.cache
def _sc_mesh():
    return plsc.VectorSubcoreMesh(
        core_axis_name="c", subcore_axis_name="s", num_cores=NC, num_subcores=NS
    )


def _deg_body(ei_hbm, zd_hbm, ones_hbm, out_hbm, idx_v, tidx_v, ones_v, zst_v,
              deg_sh, semi, sems):
    c = lax.axis_index("c")
    s = lax.axis_index("s")

    pltpu.sync_copy(zd_hbm, zst_v)
    pltpu.sync_copy(ones_hbm, ones_v)

    @pl.when(s < 10)
    def _():
        pltpu.sync_copy(zst_v, deg_sh.at[pl.ds(s * 1000, 1000)])

    plsc.subcore_barrier()

    def load(k, b):
        base = pl.multiple_of(c * E + s * EPW + k * WIN, 8)
        pltpu.async_copy(ei_hbm.at[pl.ds(base, WIN)], idx_v.at[b],
                         semi.at[b])

    def load_wait(b):
        pltpu.make_async_copy(ei_hbm.at[pl.ds(0, WIN)], idx_v.at[b],
                              semi.at[b]).wait()

    def scat_issue(b):
        pltpu.async_copy(ones_v, deg_sh.at[idx_v.at[b]], sems.at[b],
                         add=True)

    def scat_wait(b):
        pltpu.make_async_copy(ones_v, deg_sh.at[idx_v.at[b]],
                              sems.at[b]).wait()

    load(0, 0)
    load(1, 1)

    def triple(kk, carry):
        for b in (0, 1, 2):
            k = 3 * kk + b
            sp = (k + 2) % 3
            load_wait(b)
            scat_issue(b)
            if b == 0:
                pl.when(kk > 0)(lambda: scat_wait(sp))
                load(k + 2, sp)
            else:
                pl.when(kk < NFULL // 3 - 1)(lambda: scat_wait(sp))
                pl.when(kk < NFULL // 3 - 1)(lambda: load(k + 2, sp))
        return carry

    lax.fori_loop(0, NFULL // 3, triple, 0)
    scat_wait(0)
    scat_wait(1)
    scat_wait(2)
    tbase = pl.multiple_of(c * E + s * EPW + NFULL * WIN, 8)
    pltpu.sync_copy(ei_hbm.at[pl.ds(tbase, TAIL)], tidx_v.at[0])
    pltpu.sync_copy(ones_v.at[pl.ds(0, TAIL)],
                    deg_sh.at[tidx_v.at[0]], add=True)
    plsc.subcore_barrier()

    @pl.when(s < 10)
    def _():
        pltpu.sync_copy(deg_sh.at[pl.ds(s * 1000, 1000)], zst_v)
        ob = pl.multiple_of(c * N + s * 1000, 8)
        pltpu.sync_copy(zst_v, out_hbm.at[pl.ds(ob, 1000)])


@jax.jit
def _sc_degrees(edge_index):
    zd = jnp.zeros((1000,), jnp.float32)
    ones = jnp.ones((WIN,), jnp.float32)
    f = pl.kernel(
        _deg_body,
        out_type=jax.ShapeDtypeStruct((NC * N,), jnp.float32),
        mesh=_sc_mesh(),
        scratch_types=[
            pltpu.VMEM((3, WIN), jnp.int32),
            pltpu.VMEM((1, TAIL), jnp.int32),
            pltpu.VMEM((WIN,), jnp.float32),
            pltpu.VMEM((1000,), jnp.float32),
            pltpu.VMEM_SHARED((N,), jnp.float32),
            pltpu.SemaphoreType.DMA((3,)),
            pltpu.SemaphoreType.DMA((3,)),
        ],
        name="gcn_degrees",
    )
    return f(edge_index.reshape(2 * E), zd, ones)


def _trash_vec(i):
    return N + ((lax.iota(jnp.int32, 16) + i * 16) & (NT - 1))


def _scat_body(p_hbm, ei_hbm, z_hbm, out_hbm, sbuf, dbuf, rows_v, zst_v,
               m_sh, semg, semi, semd, sems):
    c = lax.axis_index("c")
    s = lax.axis_index("s")
    ebase = s * EPW

    def src_load(k, b):
        pltpu.async_copy(ei_hbm.at[pl.ds(pl.multiple_of(ebase + k * WIN, 8),
                                         WIN)],
                         sbuf.at[b], semi.at[b])

    def src_wait(b):
        pltpu.make_async_copy(ei_hbm.at[pl.ds(0, WIN)], sbuf.at[b],
                              semi.at[b]).wait()

    def add_off(b, off):
        for i in range(WIN // 16):
            sl = pl.ds(i * 16, 16)
            sbuf[b, sl] = sbuf[b, sl] + off

    def gather_issue(b):
        pltpu.async_copy(p_hbm.at[sbuf.at[b]], rows_v.at[b], semg.at[b])

    def gather_wait(b):
        pltpu.make_async_copy(p_hbm.at[sbuf.at[b]], rows_v.at[b],
                              semg.at[b]).wait()

    def dst_load(k, b):
        pltpu.async_copy(ei_hbm.at[pl.ds(pl.multiple_of(E + ebase + k * WIN,
                                                        8), WIN)],
                         dbuf.at[b], semd.at[b])

    def dst_wait(b):
        pltpu.make_async_copy(ei_hbm.at[pl.ds(0, WIN)], dbuf.at[b],
                              semd.at[b]).wait()

    def scat_issue(b):
        pltpu.async_copy(rows_v.at[b], m_sh.at[dbuf.at[b]], sems.at[b],
                         add=True)

    def scat_wait(b):
        pltpu.make_async_copy(rows_v.at[b], m_sh.at[dbuf.at[b]],
                              sems.at[b]).wait()

    for qq in range(2):
        off = (c * 2 + qq) * N

        pltpu.sync_copy(z_hbm, zst_v)

        @pl.when(s < 10)
        def _():
            for j in (0, ZR, 2 * ZR):
                n = min(ZR, 1000 - j)
                pltpu.sync_copy(zst_v.at[pl.ds(0, n)],
                                m_sh.at[pl.ds(s * 1000 + j, n)])

        @pl.when(s == 10)
        def _():
            pltpu.sync_copy(zst_v.at[pl.ds(0, NT)], m_sh.at[pl.ds(N, NT)])

        plsc.subcore_barrier()

        src_load(0, 0)
        src_wait(0)
        add_off(0, off)
        gather_issue(0)
        dst_load(0, 0)
        src_load(1, 1)

        def pair(kk, carry):
            for b in (0, 1):
                k = 2 * kk + b
                later = kk < NFULL // 2 - 1

                if b == 0:
                    pl.when(kk > 0)(lambda: scat_wait(1))
                else:
                    scat_wait(0)

                def issue_next():
                    src_wait(1 - b)
                    add_off(1 - b, off)
                    gather_issue(1 - b)

                if b == 0:
                    issue_next()
                else:
                    pl.when(later)(issue_next)
                gather_wait(b)
                pl.when(later)(lambda: src_load(k + 2, b))
                dst_wait(b)
                if b == 0:
                    dst_load(k + 1, 1)
                else:
                    pl.when(later)(lambda: dst_load(k + 1, 0))
                scat_issue(b)
            return carry

        lax.fori_loop(0, NFULL // 2, pair, 0)

        tb = pl.multiple_of(ebase + NFULL * WIN, 8)
        pltpu.sync_copy(ei_hbm.at[pl.ds(tb, TAIL)],
                        sbuf.at[0, pl.ds(0, TAIL)])
        for i in range(TAIL // 16, WIN // 16):
            sbuf[0, pl.ds(i * 16, 16)] = lax.iota(jnp.int32, 16) + i * 16
        add_off(0, off)
        pltpu.async_copy(p_hbm.at[sbuf.at[0]], rows_v.at[0], semg.at[0])
        gather_wait(0)
        tdb = pl.multiple_of(E + tb, 8)
        pltpu.sync_copy(ei_hbm.at[pl.ds(tdb, TAIL)],
                        dbuf.at[0, pl.ds(0, TAIL)])
        for i in range(TAIL // 16, WIN // 16):
            dbuf[0, pl.ds(i * 16, 16)] = _trash_vec(i)
        scat_wait(1)
        scat_issue(0)
        scat_wait(0)
        plsc.subcore_barrier()

        @pl.when(s < 10)
        def _():
            for j in (0, ZR, 2 * ZR):
                n = min(ZR, 1000 - j)
                pltpu.sync_copy(m_sh.at[pl.ds(s * 1000 + j, n)],
                                zst_v.at[pl.ds(0, n)])
                pltpu.sync_copy(zst_v.at[pl.ds(0, n)],
                                out_hbm.at[pl.ds(off + s * 1000 + j, n)])

        plsc.subcore_barrier()


@jax.jit
def _sc_scatter(p_flat, edge_index):
    z = jnp.zeros((ZR, DQ), jnp.float32)
    f = pl.kernel(
        _scat_body,
        out_type=jax.ShapeDtypeStruct((4 * N, DQ), jnp.float32),
        mesh=_sc_mesh(),
        scratch_types=[
            pltpu.VMEM((2, WIN), jnp.int32),
            pltpu.VMEM((2, WIN), jnp.int32),
            pltpu.VMEM((2, WIN, DQ), jnp.float32),
            pltpu.VMEM((ZR, DQ), jnp.float32),
            pltpu.VMEM_SHARED((N + NT, DQ), jnp.float32),
            pltpu.SemaphoreType.DMA((2,)),
            pltpu.SemaphoreType.DMA((2,)),
            pltpu.SemaphoreType.DMA((2,)),
            pltpu.SemaphoreType.DMA((2,)),
        ],
        compiler_params=pltpu.CompilerParams(use_tc_tiling_on_sc=False),
        name="gcn_edge_scatter",
    )
    return f(p_flat, edge_index.reshape(2 * E), z)


BH = BN // 2


def _csc(deg_ref):
    return lax.rsqrt(jnp.maximum(deg_ref[...], 1.0))


def _split_store(p_ref, ql, qr):
    for i in range(4):
        sl = slice(i * DQ, (i + 1) * DQ)
        p_ref[i] = jnp.concatenate([ql[:, sl], qr[:, sl]], axis=-1)


def _munpack(m_ref):
    ml = jnp.concatenate([m_ref[i][:, :DQ] for i in range(4)], axis=-1)
    mr = jnp.concatenate([m_ref[i][:, DQ:] for i in range(4)], axis=-1)
    return ml, mr


def _pre_body(x_ref, w_ref, rw_ref, rb_ref, dsl_ref, dsr_ref,
              p_ref, rl_ref, rr_ref):
    xl = x_ref[:, 0, :]
    xr = x_ref[:, 1, :]
    ql = jnp.dot(xl, w_ref[...], preferred_element_type=jnp.float32)
    qr = jnp.dot(xr, w_ref[...], preferred_element_type=jnp.float32)
    _split_store(p_ref, ql * _csc(dsl_ref), qr * _csc(dsr_ref))
    rl_ref[...] = (
        jnp.dot(xl, rw_ref[...], preferred_element_type=jnp.float32)
        + rb_ref[...]
    )
    rr_ref[...] = (
        jnp.dot(xr, rw_ref[...], preferred_element_type=jnp.float32)
        + rb_ref[...]
    )


def _mid_body(m_ref, ddl_ref, ddr_ref, b_ref, rl_ref, rr_ref, w_ref, rw_ref,
              rb_ref, dsl_ref, dsr_ref, p_ref, r2l_ref, r2r_ref):
    ml, mr = _munpack(m_ref)
    hl = jax.nn.relu(_csc(ddl_ref) * ml + b_ref[...]) + rl_ref[...]
    hr = jax.nn.relu(_csc(ddr_ref) * mr + b_ref[...]) + rr_ref[...]
    ql = jnp.dot(hl, w_ref[...], preferred_element_type=jnp.float32)
    qr = jnp.dot(hr, w_ref[...], preferred_element_type=jnp.float32)
    _split_store(p_ref, ql * _csc(dsl_ref), qr * _csc(dsr_ref))
    r2l_ref[...] = (
        jnp.dot(hl, rw_ref[...], preferred_element_type=jnp.float32)
        + rb_ref[...]
    )
    r2r_ref[...] = (
        jnp.dot(hr, rw_ref[...], preferred_element_type=jnp.float32)
        + rb_ref[...]
    )


def _mid2_body(m_ref, ddl_ref, ddr_ref, b_ref, rl_ref, rr_ref, w_ref,
               dsl_ref, dsr_ref, p_ref, hl_ref, hr_ref):
    ml, mr = _munpack(m_ref)
    hl = jax.nn.relu(_csc(ddl_ref) * ml + b_ref[...]) + rl_ref[...]
    hr = jax.nn.relu(_csc(ddr_ref) * mr + b_ref[...]) + rr_ref[...]
    ql = jnp.dot(hl, w_ref[...], preferred_element_type=jnp.float32)
    qr = jnp.dot(hr, w_ref[...], preferred_element_type=jnp.float32)
    _split_store(p_ref, ql * _csc(dsl_ref), qr * _csc(dsr_ref))
    hl_ref[...] = hl
    hr_ref[...] = hr


def _post_body(m_ref, ddl_ref, ddr_ref, b_ref, hl_ref, hr_ref, out_ref):
    ml, mr = _munpack(m_ref)
    out_ref[:, 0, :] = _csc(ddl_ref) * ml + b_ref[...] + hl_ref[...]
    out_ref[:, 1, :] = _csc(ddr_ref) * mr + b_ref[...] + hr_ref[...]


_ROW = pl.BlockSpec((BH, D), lambda i: (i, 0))
_COL = pl.BlockSpec((BH, 1), lambda i: (i, 0))
_WMAT = pl.BlockSpec((D, D), lambda i: (0, 0))
_BVEC = pl.BlockSpec((1, D), lambda i: (0, 0))
_MIN = pl.BlockSpec((4, BH, 2 * DQ), lambda i: (0, i, 0))

_P_OUT = jax.ShapeDtypeStruct((4, N // 2, 2 * DQ), jnp.float32)
_R_OUT = jax.ShapeDtypeStruct((N // 2, D), jnp.float32)


@jax.jit
def _tc_pre(x2, w, rw, rb, dsl, dsr):
    return pl.pallas_call(
        _pre_body,
        grid=(GRID,),
        in_specs=[pl.BlockSpec((BH, 2, D), lambda i: (i, 0, 0)),
                  _WMAT, _WMAT, _BVEC, _COL, _COL],
        out_specs=[_MIN, _ROW, _ROW],
        out_shape=[_P_OUT, _R_OUT, _R_OUT],
    )(x2, w, rw, rb, dsl, dsr)


@jax.jit
def _tc_mid(m, ddl, ddr, b, rl, rr, w, rw, rb, dsl, dsr):
    return pl.pallas_call(
        _mid_body,
        grid=(GRID,),
        in_specs=[_MIN, _COL, _COL, _BVEC, _ROW, _ROW, _WMAT, _WMAT, _BVEC,
                  _COL, _COL],
        out_specs=[_MIN, _ROW, _ROW],
        out_shape=[_P_OUT, _R_OUT, _R_OUT],
    )(m, ddl, ddr, b, rl, rr, w, rw, rb, dsl, dsr)


@jax.jit
def _tc_mid2(m, ddl, ddr, b, rl, rr, w, dsl, dsr):
    return pl.pallas_call(
        _mid2_body,
        grid=(GRID,),
        in_specs=[_MIN, _COL, _COL, _BVEC, _ROW, _ROW, _WMAT, _COL, _COL],
        out_specs=[_MIN, _ROW, _ROW],
        out_shape=[_P_OUT, _R_OUT, _R_OUT],
    )(m, ddl, ddr, b, rl, rr, w, dsl, dsr)


@jax.jit
def _tc_post(m, ddl, ddr, b, hl, hr):
    return pl.pallas_call(
        _post_body,
        grid=(GRID,),
        in_specs=[_MIN, _COL, _COL, _BVEC, _ROW, _ROW],
        out_specs=pl.BlockSpec((BH, 2, D), lambda i: (i, 0, 0)),
        out_shape=jax.ShapeDtypeStruct((N // 2, 2, D), jnp.float32),
    )(m, ddl, ddr, b, hl, hr)


def kernel(x, edge_index, W1, b1, W2, b2, W3, b3, rW1, rb1, rW2, rb2):
    deg = _sc_degrees(edge_index)
    dsl = deg[:N:2].reshape(N // 2, 1)
    dsr = deg[1:N:2].reshape(N // 2, 1)
    ddl = deg[N::2].reshape(N // 2, 1)
    ddr = deg[N + 1::2].reshape(N // 2, 1)
    x2 = x.reshape(N // 2, 2, D)
    b1r = b1.reshape(1, D)
    b2r = b2.reshape(1, D)
    b3r = b3.reshape(1, D)
    rb1r = rb1.reshape(1, D)
    rb2r = rb2.reshape(1, D)

    def scat(p):
        m = _sc_scatter(p.reshape(4 * N, DQ), edge_index)
        return m.reshape(4, N // 2, 2 * DQ)

    p0, r0l, r0r = _tc_pre(x2, W1, rW1, rb1r, dsl, dsr)
    m0 = scat(p0)
    p1, r1l, r1r = _tc_mid(m0, ddl, ddr, b1r, r0l, r0r, W2, rW2, rb2r,
                           dsl, dsr)
    m1 = scat(p1)
    p2, h2l, h2r = _tc_mid2(m1, ddl, ddr, b2r, r1l, r1r, W3, dsl, dsr)
    m2 = scat(p2)
    out = _tc_post(m2, ddl, ddr, b3r, h2l, h2r)
    return out.reshape(N, D)
